# Initial kernel scaffold; baseline (speedup 1.0000x reference)
import jax
import jax.numpy as jnp
from jax.experimental import pallas as pl


def kernel(mem_img, mem_gps, mem_coords, img_emb, gps_emb, gps_coords, ptr):
    raise NotImplementedError("write your pallas kernel here")


# speedup vs baseline: 1.4208x; 1.4208x over previous
import functools

import jax
import jax.numpy as jnp
from jax import lax
from jax.experimental import pallas as pl
from jax.experimental.pallas import tpu as pltpu
from jax.experimental.pallas import tpu_sc as plsc

NUM_CORES = 2
NUM_SUBCORES = 16
NW = NUM_CORES * NUM_SUBCORES
CH = 64
PAD = CH + 8


def kernel(mem_img, mem_gps, mem_coords, img_emb, gps_emb, gps_coords, ptr):
    M, D = mem_img.shape
    B = img_emb.shape[0]
    C = mem_coords.shape[1]
    W = 2 * D + C
    rows_per_w = M // NW
    n_chunks = rows_per_w // CH

    p0 = jnp.asarray(ptr, jnp.int32) % jnp.int32(M)
    ptr_vec = jnp.full((16,), p0, dtype=jnp.int32)
    CP = 16
    ie_p = jnp.pad(img_emb, ((0, PAD), (0, 0)))
    ge_p = jnp.pad(gps_emb, ((0, PAD), (0, 0)))
    gc_p = jnp.pad(gps_coords, ((0, PAD), (0, CP - C)))

    mesh = plsc.VectorSubcoreMesh(core_axis_name="c", subcore_axis_name="s")

    @functools.partial(
        pl.kernel,
        out_type=jax.ShapeDtypeStruct((M, W), jnp.float32),
        mesh=mesh,
        compiler_params=pltpu.CompilerParams(use_tc_tiling_on_sc=False),
        scratch_types=[
            pltpu.VMEM((CH, D), jnp.float32),
            pltpu.VMEM((CH, D), jnp.float32),
            pltpu.VMEM((CH, CP), jnp.float32),
            pltpu.VMEM((PAD, D), jnp.float32),
            pltpu.VMEM((PAD, CP), jnp.float32),
            pltpu.VMEM((16,), jnp.int32),
        ],
    )
    def run(mi, mg, mc, ie, ge, gc, pv, out, bimg, bgps, bcrd, spill, spill_c,
            pbuf):
        wid = lax.axis_index("s") * NUM_CORES + lax.axis_index("c")
        base = wid * rows_per_w
        pltpu.sync_copy(pv, pbuf)
        p = pbuf[...][0]

        def chunk_body(t, carry):
            c0 = pl.multiple_of(base + t * CH, CH)
            pltpu.sync_copy(mi.at[pl.ds(c0, CH)], bimg)
            pltpu.sync_copy(mg.at[pl.ds(c0, CH)], bgps)
            pltpu.sync_copy(mc.at[pl.ds(c0, CH)], bcrd.at[:, pl.ds(0, C)])

            d = c0 - p
            s = jnp.where(d < 0, d + M, d)
            full = s <= B - CH
            aligned = lax.bitwise_and(s, 7) == 0

            @pl.when(jnp.logical_and(full, aligned))
            def _fast_overlay():
                sa = pl.multiple_of(s, 8)
                pltpu.sync_copy(ie.at[pl.ds(sa, CH)], bimg)
                pltpu.sync_copy(ge.at[pl.ds(sa, CH)], bgps)
                pltpu.sync_copy(gc.at[pl.ds(sa, CH)], bcrd)

            @pl.when(jnp.logical_and(jnp.logical_or(s < B, s > M - CH),
                                     jnp.logical_not(
                                         jnp.logical_and(full, aligned))))
            def _slow_overlay():
                head = s < B
                jlo = jnp.where(head, 0, M - s)
                jhi = jnp.where(head, jnp.minimum(CH, B - s), CH)
                e0 = jnp.where(head, s, 0)
                a = pl.multiple_of(lax.bitwise_and(e0, -8), 8)
                off = e0 - a - jlo
                pltpu.sync_copy(ie.at[pl.ds(a, PAD)], spill)
                def row_i(j, c):
                    for g in range(D // 16):
                        bimg[j, pl.ds(g * 16, 16)] = (
                            spill[j + off, pl.ds(g * 16, 16)])
                    return c
                lax.fori_loop(jlo, jhi, row_i, 0)
                pltpu.sync_copy(ge.at[pl.ds(a, PAD)], spill)
                def row_g(j, c):
                    for g in range(D // 16):
                        bgps[j, pl.ds(g * 16, 16)] = (
                            spill[j + off, pl.ds(g * 16, 16)])
                    return c
                lax.fori_loop(jlo, jhi, row_g, 0)
                pltpu.sync_copy(gc.at[pl.ds(a, PAD)], spill_c)
                def row_c(j, c):
                    bcrd[j, :] = spill_c[j + off, :]
                    return c
                lax.fori_loop(jlo, jhi, row_c, 0)

            pltpu.sync_copy(bimg, out.at[pl.ds(c0, CH), pl.ds(0, D)])
            pltpu.sync_copy(bgps, out.at[pl.ds(c0, CH), pl.ds(D, D)])
            pltpu.sync_copy(bcrd.at[:, pl.ds(0, C)],
                            out.at[pl.ds(c0, CH), pl.ds(2 * D, C)])
            return carry

        lax.fori_loop(0, n_chunks, chunk_body, 0)

    return run(mem_img, mem_gps, mem_coords, ie_p, ge_p, gc_p, ptr_vec)
